# baseline (speedup 1.0000x reference)
import functools

import jax
import jax.numpy as jnp
from jax import lax
from jax.experimental import pallas as pl
from jax.experimental.pallas import tpu as pltpu
from jax.experimental.pallas import tpu_sc as plsc

B = 4096
C = 20
N_NEGS = 20
D = 32
NUM_WORKERS = 32
EPW = B // NUM_WORKERS
NROW = C + C * N_NEGS
ROWS = 432
NBLK = ROWS // 16
NBUF = 4


def _sc_scores(iv_all, idx_flat, w_o):
  mesh = plsc.VectorSubcoreMesh(core_axis_name="c", subcore_axis_name="s")

  @functools.partial(
      pl.kernel,
      out_type=jax.ShapeDtypeStruct((B * ROWS,), jnp.float32),
      mesh=mesh,
      compiler_params=pltpu.CompilerParams(
          needs_layout_passes=False, use_tc_tiling_on_sc=False),
      scratch_types=[
          pltpu.VMEM((EPW, D), jnp.float32),
          pltpu.VMEM((EPW * ROWS,), jnp.int32),
          *([pltpu.VMEM((ROWS, D), jnp.float32)] * NBUF),
          *([pltpu.VMEM((ROWS,), jnp.float32)] * NBUF),
          *([pltpu.SemaphoreType.DMA] * NBUF),
          *([pltpu.SemaphoreType.DMA] * NBUF),
      ],
  )
  def kern(iv_hbm, idx_hbm, wo_hbm, out_hbm, iv_v, idx_v,
           rb0, rb1, rb2, rb3, sc0, sc1, sc2, sc3,
           g0, g1, g2, g3, o0, o1, o2, o3):
    rbs = (rb0, rb1, rb2, rb3)
    scs = (sc0, sc1, sc2, sc3)
    gsems = (g0, g1, g2, g3)
    osems = (o0, o1, o2, o3)
    cid = lax.axis_index("c")
    sid = lax.axis_index("s")
    wid = sid * 2 + cid
    base = wid * EPW

    pltpu.sync_copy(iv_hbm.at[pl.ds(base, EPW)], iv_v)
    pltpu.sync_copy(idx_hbm.at[pl.ds(base * ROWS, EPW * ROWS)], idx_v)

    lane = lax.iota(jnp.int32, 16)

    chunks = ((0, 128), (128, 128), (256, 128), (384, NROW - 384))

    def fire(e, rb, sem):
      off = e * ROWS
      for (o, n) in chunks:
        pltpu.async_copy(wo_hbm.at[idx_v.at[pl.ds(off + o, n)]],
                         rb.at[pl.ds(o, n)], sem)

    def drain_gather(rb, sem):
      pltpu.make_async_copy(
          wo_hbm.at[pl.ds(0, NROW)], rb.at[pl.ds(0, NROW)], sem).wait()

    def drain_score(sc, sem):
      pltpu.make_async_copy(sc, out_hbm.at[pl.ds(0, ROWS)], sem).wait()

    def compute(e, rb, sc):
      iv0 = iv_v[e, 0:16]
      iv1 = iv_v[e, 16:32]

      def blk_body(blk, carry):
        sv = jnp.zeros((16,), jnp.float32)
        for j in range(16):
          r0 = rb[blk * 16 + j, 0:16]
          r1 = rb[blk * 16 + j, 16:32]
          s = jnp.sum(r0 * iv0 + r1 * iv1)
          sv = jnp.where(lane == j, s, sv)
        sc[pl.ds(blk * 16, 16)] = sv
        return carry

      lax.fori_loop(0, NBLK, blk_body, 0)

    for k in range(NBUF):
      fire(k, rbs[k], gsems[k])

    def loop_body(i, carry):
      e0 = i * NBUF
      for k in range(NBUF):
        e = e0 + k
        drain_gather(rbs[k], gsems[k])

        @pl.when(i >= 1)
        def _():
          drain_score(scs[k], osems[k])

        compute(e, rbs[k], scs[k])

        @pl.when(e + NBUF < EPW)
        def _():
          fire(e + NBUF, rbs[k], gsems[k])

        pltpu.async_copy(scs[k], out_hbm.at[pl.ds((base + e) * ROWS, ROWS)],
                         osems[k])
      return carry

    lax.fori_loop(0, EPW // NBUF, loop_body, 0)
    for k in range(NBUF):
      drain_score(scs[k], osems[k])

  return kern(iv_all, idx_flat, w_o)


BC = 1024


def _tc_detile(w):
  v = w.shape[0]
  grid = pl.cdiv(v, BC)

  def kern(x_ref, o_ref):
    eye = (lax.broadcasted_iota(jnp.int32, (D, D), 0)
           == lax.broadcasted_iota(jnp.int32, (D, D), 1)).astype(jnp.float32)
    y = lax.dot_general(x_ref[...], eye, (((0,), (0,)), ((), ())),
                        preferred_element_type=jnp.float32)
    o_ref[...] = jnp.concatenate(
        [y[256 * u:256 * (u + 1), :] for u in range(4)], axis=1)

  return pl.pallas_call(
      kern,
      grid=(grid,),
      in_specs=[pl.BlockSpec((D, BC), lambda j: (0, j))],
      out_specs=pl.BlockSpec((BC // 4, 128), lambda j: (j, 0)),
      out_shape=jax.ShapeDtypeStruct((grid * (BC // 4), 128), jnp.float32),
  )(w.T)


def _permute_idx(r):
  return (r & ~(BC - 1)) + ((r & 255) << 2) + ((r >> 8) & 3)


def _tc_loss(scores_flat):
  rows = (B * ROWS) // 128

  def kern(s_ref, o_ref):
    s = s_ref[...]
    p = (lax.broadcasted_iota(jnp.int32, (rows, 128), 0) * 128
         + lax.broadcasted_iota(jnp.int32, (rows, 128), 1))
    col = lax.rem(p, ROWS)
    x = jnp.where(col < C, -s, s)
    sp = jnp.maximum(x, 0.0) + jnp.log(1.0 + jnp.exp(-jnp.abs(x)))
    term = jnp.where(col < NROW, sp, 0.0)
    o_ref[0, 0] = jnp.sum(term) * (1.0 / (B * C))

  out = pl.pallas_call(
      kern,
      out_shape=jax.ShapeDtypeStruct((1, 1), jnp.float32),
      in_specs=[pl.BlockSpec(memory_space=pltpu.VMEM)],
      out_specs=pl.BlockSpec(memory_space=pltpu.SMEM),
  )(scores_flat.reshape(rows, 128))
  return out[0, 0]


def kernel(iword, owords, W_i, W_o):
  batch_size = iword.shape[0]
  context_size = owords.shape[1]
  vocab = W_o.shape[0]
  nwords = jax.random.randint(
      jax.random.key(42), (batch_size, context_size * N_NEGS), 0, vocab - 1)
  pad = jnp.zeros((batch_size, ROWS - NROW), jnp.int32)
  idx = jnp.concatenate(
      [owords.astype(jnp.int32), nwords.astype(jnp.int32), pad], axis=1)
  iv_all = jnp.take(W_i.astype(jnp.float32), iword, axis=0)
  w_lin = _tc_detile(W_o.astype(jnp.float32))
  w_lin = w_lin.reshape(w_lin.shape[0] * (128 // D), D)
  scores = _sc_scores(iv_all, _permute_idx(idx).reshape(-1), w_lin)
  return _tc_loss(scores)


# speedup vs baseline: 1.2126x; 1.2126x over previous
import functools

import jax
import jax.numpy as jnp
from jax import lax
from jax.experimental import pallas as pl
from jax.experimental.pallas import tpu as pltpu
from jax.experimental.pallas import tpu_sc as plsc

B = 4096
C = 20
N_NEGS = 20
D = 32
NUM_WORKERS = 32
EPW = B // NUM_WORKERS
NROW = C + C * N_NEGS
ROWS = 432
NBLK = ROWS // 16
NBUF = 4


def _sc_scores(iv_all, idx_flat, w_o):
  mesh = plsc.VectorSubcoreMesh(core_axis_name="c", subcore_axis_name="s")

  @functools.partial(
      pl.kernel,
      out_type=jax.ShapeDtypeStruct((B * ROWS,), jnp.float32),
      mesh=mesh,
      compiler_params=pltpu.CompilerParams(
          needs_layout_passes=False, use_tc_tiling_on_sc=False),
      scratch_types=[
          pltpu.VMEM((EPW, D), jnp.float32),
          pltpu.VMEM((EPW * ROWS,), jnp.int32),
          *([pltpu.VMEM((ROWS, D), jnp.bfloat16)] * NBUF),
          *([pltpu.VMEM((ROWS,), jnp.float32)] * NBUF),
          *([pltpu.SemaphoreType.DMA] * NBUF),
          *([pltpu.SemaphoreType.DMA] * NBUF),
      ],
  )
  def kern(iv_hbm, idx_hbm, wo_hbm, out_hbm, iv_v, idx_v,
           rb0, rb1, rb2, rb3, sc0, sc1, sc2, sc3,
           g0, g1, g2, g3, o0, o1, o2, o3):
    rbs = (rb0, rb1, rb2, rb3)
    scs = (sc0, sc1, sc2, sc3)
    gsems = (g0, g1, g2, g3)
    osems = (o0, o1, o2, o3)
    cid = lax.axis_index("c")
    sid = lax.axis_index("s")
    wid = sid * 2 + cid
    base = wid * EPW

    pltpu.sync_copy(iv_hbm.at[pl.ds(base, EPW)], iv_v)
    pltpu.sync_copy(idx_hbm.at[pl.ds(base * ROWS, EPW * ROWS)], idx_v)

    lane = lax.iota(jnp.int32, 16)

    chunks = ((0, 128), (128, 128), (256, 128), (384, NROW - 384))

    def fire(e, rb, sem):
      off = e * ROWS
      for (o, n) in chunks:
        pltpu.async_copy(wo_hbm.at[idx_v.at[pl.ds(off + o, n)]],
                         rb.at[pl.ds(o, n)], sem)

    def drain_gather(rb, sem):
      pltpu.make_async_copy(
          wo_hbm.at[pl.ds(0, NROW)], rb.at[pl.ds(0, NROW)], sem).wait()

    def drain_score(sc, sem):
      pltpu.make_async_copy(sc, out_hbm.at[pl.ds(0, ROWS)], sem).wait()

    def compute(e, rb, sc):
      iv0 = iv_v[e, 0:16]
      iv1 = iv_v[e, 16:32]

      def blk_body(blk, carry):
        sv = jnp.zeros((16,), jnp.float32)
        for j in range(16):
          row = rb[blk * 16 + j, :]
          a, b = plsc.unpack(row, format=plsc.PackFormat.INTERLEAVED)
          s = jnp.sum(a * iv0 + b * iv1)
          sv = jnp.where(lane == j, s, sv)
        sc[pl.ds(blk * 16, 16)] = sv
        return carry

      lax.fori_loop(0, NBLK, blk_body, 0)

    for k in range(NBUF):
      fire(k, rbs[k], gsems[k])

    def loop_body(i, carry):
      e0 = i * NBUF
      for k in range(NBUF):
        e = e0 + k
        drain_gather(rbs[k], gsems[k])

        @pl.when(i >= 1)
        def _():
          drain_score(scs[k], osems[k])

        compute(e, rbs[k], scs[k])

        @pl.when(e + NBUF < EPW)
        def _():
          fire(e + NBUF, rbs[k], gsems[k])

        pltpu.async_copy(scs[k], out_hbm.at[pl.ds((base + e) * ROWS, ROWS)],
                         osems[k])
      return carry

    lax.fori_loop(0, EPW // NBUF, loop_body, 0)
    for k in range(NBUF):
      drain_score(scs[k], osems[k])

  return kern(iv_all, idx_flat, w_o)


def _tc_loss(scores_flat):
  rows = (B * ROWS) // 128

  def kern(s_ref, o_ref):
    s = s_ref[...]
    p = (lax.broadcasted_iota(jnp.int32, (rows, 128), 0) * 128
         + lax.broadcasted_iota(jnp.int32, (rows, 128), 1))
    col = lax.rem(p, ROWS)
    x = jnp.where(col < C, -s, s)
    sp = jnp.maximum(x, 0.0) + jnp.log(1.0 + jnp.exp(-jnp.abs(x)))
    term = jnp.where(col < NROW, sp, 0.0)
    o_ref[0, 0] = jnp.sum(term) * (1.0 / (B * C))

  out = pl.pallas_call(
      kern,
      out_shape=jax.ShapeDtypeStruct((1, 1), jnp.float32),
      in_specs=[pl.BlockSpec(memory_space=pltpu.VMEM)],
      out_specs=pl.BlockSpec(memory_space=pltpu.SMEM),
  )(scores_flat.reshape(rows, 128))
  return out[0, 0]


def kernel(iword, owords, W_i, W_o):
  batch_size = iword.shape[0]
  context_size = owords.shape[1]
  vocab = W_o.shape[0]
  nwords = jax.random.randint(
      jax.random.key(42), (batch_size, context_size * N_NEGS), 0, vocab - 1)
  pad = jnp.zeros((batch_size, ROWS - NROW), jnp.int32)
  idx = jnp.concatenate(
      [owords.astype(jnp.int32), nwords.astype(jnp.int32), pad], axis=1)
  iv_all = jnp.take(W_i.astype(jnp.float32), iword, axis=0)
  iv_perm = jnp.concatenate([iv_all[:, 0::2], iv_all[:, 1::2]], axis=1)
  scores = _sc_scores(iv_perm, idx.reshape(-1), W_o.astype(jnp.bfloat16))
  return _tc_loss(scores)


# speedup vs baseline: 1.2976x; 1.0701x over previous
import functools

import jax
import jax.numpy as jnp
from jax import lax
from jax.experimental import pallas as pl
from jax.experimental.pallas import tpu as pltpu
from jax.experimental.pallas import tpu_sc as plsc

B = 4096
C = 20
N_NEGS = 20
D = 32
NUM_WORKERS = 32
EPW = B // NUM_WORKERS
NROW = C + C * N_NEGS
ROWS = 432
NBLK = ROWS // 16
NBUF = 4


def _sc_scores(iv_all, idx_flat, w_o):
  mesh = plsc.VectorSubcoreMesh(core_axis_name="c", subcore_axis_name="s")

  @functools.partial(
      pl.kernel,
      out_type=jax.ShapeDtypeStruct((B * ROWS,), jnp.float32),
      mesh=mesh,
      compiler_params=pltpu.CompilerParams(
          needs_layout_passes=False, use_tc_tiling_on_sc=False),
      scratch_types=[
          pltpu.VMEM((EPW, D), jnp.float32),
          pltpu.VMEM((EPW * ROWS,), jnp.int32),
          *([pltpu.VMEM((ROWS, D), jnp.float32)] * NBUF),
          *([pltpu.VMEM((ROWS,), jnp.float32)] * NBUF),
          *([pltpu.SemaphoreType.DMA] * NBUF),
          *([pltpu.SemaphoreType.DMA] * NBUF),
      ],
  )
  def kern(iv_hbm, idx_hbm, wo_hbm, out_hbm, iv_v, idx_v,
           rb0, rb1, rb2, rb3, sc0, sc1, sc2, sc3,
           g0, g1, g2, g3, o0, o1, o2, o3):
    rbs = (rb0, rb1, rb2, rb3)
    scs = (sc0, sc1, sc2, sc3)
    gsems = (g0, g1, g2, g3)
    osems = (o0, o1, o2, o3)
    cid = lax.axis_index("c")
    sid = lax.axis_index("s")
    wid = sid * 2 + cid
    base = wid * EPW

    pltpu.sync_copy(iv_hbm.at[pl.ds(base, EPW)], iv_v)
    pltpu.sync_copy(idx_hbm.at[pl.ds(base * ROWS, EPW * ROWS)], idx_v)

    lane = lax.iota(jnp.int32, 16)

    chunks = ((0, 128), (128, 128), (256, 128), (384, NROW - 384))

    def fire(e, rb, sem):
      off = e * ROWS
      for (o, n) in chunks:
        pltpu.async_copy(wo_hbm.at[idx_v.at[pl.ds(off + o, n)]],
                         rb.at[pl.ds(o, n)], sem)

    def drain_gather(rb, sem):
      pltpu.make_async_copy(
          wo_hbm.at[pl.ds(0, NROW)], rb.at[pl.ds(0, NROW)], sem).wait()

    def drain_score(sc, sem):
      pltpu.make_async_copy(sc, out_hbm.at[pl.ds(0, ROWS)], sem).wait()

    def compute(e, rb, sc):
      iv0 = iv_v[e, 0:16]
      iv1 = iv_v[e, 16:32]

      def blk_body(blk, carry):
        sv = jnp.zeros((16,), jnp.float32)
        for j in range(16):
          r0 = rb[blk * 16 + j, 0:16]
          r1 = rb[blk * 16 + j, 16:32]
          s = jnp.sum(r0 * iv0 + r1 * iv1)
          sv = jnp.where(lane == j, s, sv)
        sc[pl.ds(blk * 16, 16)] = sv
        return carry

      lax.fori_loop(0, NBLK, blk_body, 0)

    for k in range(NBUF):
      fire(k, rbs[k], gsems[k])

    def loop_body(i, carry):
      e0 = i * NBUF
      for k in range(NBUF):
        e = e0 + k
        drain_gather(rbs[k], gsems[k])

        @pl.when(i >= 1)
        def _():
          drain_score(scs[k], osems[k])

        compute(e, rbs[k], scs[k])

        @pl.when(e + NBUF < EPW)
        def _():
          fire(e + NBUF, rbs[k], gsems[k])

        pltpu.async_copy(scs[k], out_hbm.at[pl.ds((base + e) * ROWS, ROWS)],
                         osems[k])
      return carry

    lax.fori_loop(0, EPW // NBUF, loop_body, 0)
    for k in range(NBUF):
      drain_score(scs[k], osems[k])

  return kern(iv_all, idx_flat, w_o)


def _tc_loss(scores_flat):
  rows = (B * ROWS) // 128

  def kern(s_ref, o_ref):
    s = s_ref[...]
    p = (lax.broadcasted_iota(jnp.int32, (rows, 128), 0) * 128
         + lax.broadcasted_iota(jnp.int32, (rows, 128), 1))
    col = lax.rem(p, ROWS)
    x = jnp.where(col < C, -s, s)
    sp = jnp.maximum(x, 0.0) + jnp.log(1.0 + jnp.exp(-jnp.abs(x)))
    term = jnp.where(col < NROW, sp, 0.0)
    o_ref[0, 0] = jnp.sum(term) * (1.0 / (B * C))

  out = pl.pallas_call(
      kern,
      out_shape=jax.ShapeDtypeStruct((1, 1), jnp.float32),
      in_specs=[pl.BlockSpec(memory_space=pltpu.VMEM)],
      out_specs=pl.BlockSpec(memory_space=pltpu.SMEM),
  )(scores_flat.reshape(rows, 128))
  return out[0, 0]


def kernel(iword, owords, W_i, W_o):
  batch_size = iword.shape[0]
  context_size = owords.shape[1]
  vocab = W_o.shape[0]
  nwords = jax.random.randint(
      jax.random.key(42), (batch_size, context_size * N_NEGS), 0, vocab - 1)
  pad = jnp.zeros((batch_size, ROWS - NROW), jnp.int32)
  idx = jnp.concatenate(
      [owords.astype(jnp.int32), nwords.astype(jnp.int32), pad], axis=1)
  iv_all = jnp.take(W_i.astype(jnp.float32), iword, axis=0)
  scores = _sc_scores(iv_all, idx.reshape(-1), W_o.astype(jnp.float32))
  return _tc_loss(scores)
